# Initial kernel scaffold; baseline (speedup 1.0000x reference)
import jax
import jax.numpy as jnp
from jax.experimental import pallas as pl


def kernel(cos_cached, sin_cached, position_ids):
    raise NotImplementedError("write your pallas kernel here")


# speedup vs baseline: 3.5321x; 3.5321x over previous
import functools

import jax
import jax.numpy as jnp
from jax import lax
from jax.experimental import pallas as pl
from jax.experimental.pallas import tpu as pltpu
from jax.experimental.pallas import tpu_sc as plsc

DIM = 128
NC = 2
NS = 16
NW = NC * NS
TOTAL = 4 * 4096
BPW = TOTAL // NW
CH = 128
NCH = BPW // CH

_mesh = plsc.VectorSubcoreMesh(core_axis_name="c", subcore_axis_name="s")


@functools.partial(
    pl.kernel,
    mesh=_mesh,
    out_type=(
        jax.ShapeDtypeStruct((TOTAL, DIM), jnp.float32),
        jax.ShapeDtypeStruct((TOTAL, DIM), jnp.float32),
    ),
    scratch_types=[
        pltpu.VMEM((NCH, CH), jnp.int32),
        pltpu.VMEM((CH, DIM), jnp.float32),
        pltpu.VMEM((CH, DIM), jnp.float32),
        pltpu.SemaphoreType.DMA,
        pltpu.SemaphoreType.DMA,
    ],
)
def _gather_kernel(cos_hbm, sin_hbm, pos_hbm, cos_out, sin_out,
                   idx_v, cbuf, sbuf, csem, ssem):
    wid = lax.axis_index("s") * NC + lax.axis_index("c")
    base = wid * BPW
    pltpu.sync_copy(pos_hbm.at[wid], idx_v)
    for c in range(NCH):
        ccopy = pltpu.async_copy(cos_hbm.at[idx_v.at[c]], cbuf, csem)
        scopy = pltpu.async_copy(sin_hbm.at[idx_v.at[c]], sbuf, ssem)
        ccopy.wait()
        pltpu.sync_copy(cbuf, cos_out.at[pl.ds(base + c * CH, CH)])
        scopy.wait()
        pltpu.sync_copy(sbuf, sin_out.at[pl.ds(base + c * CH, CH)])


def kernel(cos_cached, sin_cached, position_ids):
    b, s = position_ids.shape
    pos = position_ids.astype(jnp.int32).reshape(NW, NCH, CH)
    cos_flat, sin_flat = _gather_kernel(cos_cached, sin_cached, pos)
    return (cos_flat.reshape(b, s, DIM), sin_flat.reshape(b, s, DIM))


# speedup vs baseline: 3.7485x; 1.0613x over previous
import functools

import jax
import jax.numpy as jnp
from jax import lax
from jax.experimental import pallas as pl
from jax.experimental.pallas import tpu as pltpu
from jax.experimental.pallas import tpu_sc as plsc

DIM = 128
NC = 2
NS = 16
NW = NC * NS
TOTAL = 4 * 4096
BPW = TOTAL // NW
CH = 128
NCH = BPW // CH

_mesh = plsc.VectorSubcoreMesh(core_axis_name="c", subcore_axis_name="s")


@functools.partial(
    pl.kernel,
    mesh=_mesh,
    out_type=(
        jax.ShapeDtypeStruct((TOTAL, DIM), jnp.float32),
        jax.ShapeDtypeStruct((TOTAL, DIM), jnp.float32),
    ),
    scratch_types=[
        pltpu.VMEM((NCH, CH), jnp.int32),
        pltpu.VMEM((3, CH, DIM), jnp.float32),
        pltpu.VMEM((3, CH, DIM), jnp.float32),
        pltpu.SemaphoreType.DMA,
        pltpu.SemaphoreType.DMA,
        pltpu.SemaphoreType.DMA,
        pltpu.SemaphoreType.DMA,
    ],
)
def _gather_kernel(cos_hbm, sin_hbm, pos_hbm, cos_out, sin_out,
                   idx_v, cbuf, sbuf, gcsem, gssem, wcsem, wssem):
    wid = lax.axis_index("s") * NC + lax.axis_index("c")
    base = wid * BPW
    pltpu.sync_copy(pos_hbm.at[wid], idx_v)
    gc, gs, wc, ws = {}, {}, {}, {}
    for c in range(min(3, NCH)):
        gc[c] = pltpu.async_copy(cos_hbm.at[idx_v.at[c]], cbuf.at[c % 3], gcsem)
        gs[c] = pltpu.async_copy(sin_hbm.at[idx_v.at[c]], sbuf.at[c % 3], gssem)
    for c in range(NCH):
        out = pl.ds(base + c * CH, CH)
        gc[c].wait()
        wc[c] = pltpu.async_copy(cbuf.at[c % 3], cos_out.at[out], wcsem)
        gs[c].wait()
        ws[c] = pltpu.async_copy(sbuf.at[c % 3], sin_out.at[out], wssem)
        nxt = c + 3
        if nxt < NCH:
            wc[c].wait()
            gc[nxt] = pltpu.async_copy(cos_hbm.at[idx_v.at[nxt]], cbuf.at[nxt % 3], gcsem)
            ws[c].wait()
            gs[nxt] = pltpu.async_copy(sin_hbm.at[idx_v.at[nxt]], sbuf.at[nxt % 3], gssem)
            del wc[c], ws[c]
    for c in wc:
        wc[c].wait()
    for c in ws:
        ws[c].wait()


def kernel(cos_cached, sin_cached, position_ids):
    b, s = position_ids.shape
    pos = position_ids.astype(jnp.int32).reshape(NW, NCH, CH)
    cos_flat, sin_flat = _gather_kernel(cos_cached, sin_cached, pos)
    return (cos_flat.reshape(b, s, DIM), sin_flat.reshape(b, s, DIM))
